# baseline (speedup 1.0000x reference)
import functools

import jax
import jax.numpy as jnp
from jax import lax
from jax.experimental import pallas as pl
from jax.experimental.pallas import tpu as pltpu
from jax.experimental.pallas import tpu_sc as plsc

D_MODEL = 64
SCALE = 8.0
NUM_CORES = 2
NUM_SUBCORES = 16
NUM_WORKERS = NUM_CORES * NUM_SUBCORES
LANES = 16


@functools.partial(jax.jit, static_argnums=(2, 3))
def _emb_lookup(idx_flat, table, n_rows, chunk):
    rows_per_w = n_rows // NUM_WORKERS
    n_chunks = rows_per_w // chunk
    mesh = plsc.VectorSubcoreMesh(core_axis_name="c", subcore_axis_name="s")

    @functools.partial(
        pl.kernel,
        mesh=mesh,
        out_type=jax.ShapeDtypeStruct((n_rows, D_MODEL), jnp.float32),
        scratch_types=[
            pltpu.VMEM((rows_per_w,), jnp.int32),
            pltpu.VMEM((chunk, D_MODEL), jnp.float32),
            pltpu.SemaphoreType.DMA,
        ],
        compiler_params=pltpu.CompilerParams(use_tc_tiling_on_sc=False),
    )
    def k(idx_hbm, table_hbm, out_hbm, idx_v, rows_v, sem):
        wid = lax.axis_index("s") * NUM_CORES + lax.axis_index("c")
        base = wid * rows_per_w
        pltpu.sync_copy(idx_hbm.at[pl.ds(base, rows_per_w)], idx_v)

        def chunk_body(g, carry):
            idx_slice = idx_v.at[pl.ds(g * chunk, chunk)]
            pltpu.async_copy(table_hbm.at[idx_slice], rows_v, sem).wait()

            def row_body(r, c2):
                for kk in range(D_MODEL // LANES):
                    sl = pl.ds(kk * LANES, LANES)
                    rows_v[r, sl] = rows_v[r, sl] * SCALE
                return c2

            lax.fori_loop(0, chunk, row_body, 0, unroll=2)
            pltpu.sync_copy(rows_v, out_hbm.at[pl.ds(base + g * chunk, chunk)])
            return carry

        lax.fori_loop(0, n_chunks, chunk_body, 0)

    return k(idx_flat, table)


def kernel(x, table):
    b, l = x.shape
    n_rows = b * l
    idx_flat = x.reshape(n_rows).astype(jnp.int32)
    out = _emb_lookup(idx_flat, table, n_rows, 512)
    return out.reshape(b, l, D_MODEL)


# speedup vs baseline: 1.0716x; 1.0716x over previous
import functools

import jax
import jax.numpy as jnp
from jax import lax
from jax.experimental import pallas as pl
from jax.experimental.pallas import tpu as pltpu
from jax.experimental.pallas import tpu_sc as plsc

D_MODEL = 64
SCALE = 8.0
NUM_CORES = 2
NUM_SUBCORES = 16
NUM_WORKERS = NUM_CORES * NUM_SUBCORES
LANES = 16
NBUF = 4
LOOKAHEAD = 2


@functools.partial(jax.jit, static_argnums=(2, 3))
def _emb_lookup(idx_flat, table, n_rows, chunk):
    rows_per_w = n_rows // NUM_WORKERS
    n_chunks = rows_per_w // chunk
    assert n_chunks % NBUF == 0 and n_chunks >= 2 * NBUF
    mesh = plsc.VectorSubcoreMesh(core_axis_name="c", subcore_axis_name="s")

    @functools.partial(
        pl.kernel,
        mesh=mesh,
        out_type=jax.ShapeDtypeStruct((n_rows, D_MODEL), jnp.float32),
        scratch_types=[
            pltpu.VMEM((rows_per_w,), jnp.int32),
            [pltpu.VMEM((chunk, D_MODEL), jnp.float32) for _ in range(NBUF)],
            [pltpu.SemaphoreType.DMA for _ in range(NBUF)],
            [pltpu.SemaphoreType.DMA for _ in range(NBUF)],
        ],
        compiler_params=pltpu.CompilerParams(use_tc_tiling_on_sc=False),
    )
    def k(idx_hbm, table_hbm, out_hbm, idx_v, bufs, sem_g, sem_w):
        wid = lax.axis_index("s") * NUM_CORES + lax.axis_index("c")
        base = wid * rows_per_w
        pltpu.sync_copy(idx_hbm.at[pl.ds(base, rows_per_w)], idx_v)

        def fire_gather(c, b):
            pltpu.async_copy(
                table_hbm.at[idx_v.at[pl.ds(c * chunk, chunk)]], bufs[b], sem_g[b]
            )

        def wait_gather(c, b):
            pltpu.make_async_copy(
                table_hbm.at[idx_v.at[pl.ds(c * chunk, chunk)]], bufs[b], sem_g[b]
            ).wait()

        def fire_write(c, b):
            pltpu.async_copy(
                bufs[b], out_hbm.at[pl.ds(base + c * chunk, chunk)], sem_w[b]
            )

        def wait_write(c, b):
            pltpu.make_async_copy(
                bufs[b], out_hbm.at[pl.ds(base + c * chunk, chunk)], sem_w[b]
            ).wait()

        for j in range(LOOKAHEAD):
            fire_gather(j, j)

        def round_body(r, carry):
            for j in range(NBUF):
                g = r * NBUF + j
                wait_gather(g, j)

                @plsc.parallel_loop(0, chunk, 1, unroll=8)
                def _scale(row):
                    for kk in range(D_MODEL // LANES):
                        sl = pl.ds(kk * LANES, LANES)
                        bufs[j][row, sl] = bufs[j][row, sl] * SCALE

                fire_write(g, j)

                nb = (j + LOOKAHEAD) % NBUF

                @pl.when(g + LOOKAHEAD < n_chunks)
                def _():
                    @pl.when(g >= NBUF - LOOKAHEAD)
                    def _():
                        wait_write(g + LOOKAHEAD - NBUF, nb)

                    fire_gather(g + LOOKAHEAD, nb)

            return carry

        lax.fori_loop(0, n_chunks // NBUF, round_body, 0)

        for j in range(NBUF):
            wait_write(n_chunks - NBUF + j, (n_chunks - NBUF + j) % NBUF)

    return k(idx_flat, table)


def kernel(x, table):
    b, l = x.shape
    n_rows = b * l
    idx_flat = x.reshape(n_rows).astype(jnp.int32)
    out = _emb_lookup(idx_flat, table, n_rows, 320)
    return out.reshape(b, l, D_MODEL)
